# Initial kernel scaffold; baseline (speedup 1.0000x reference)
import jax
import jax.numpy as jnp
from jax.experimental import pallas as pl


def kernel(price_tensor, category, sub_category, element, brand, product_id_remapped, edge_index, batch, cat_emb, sub_emb, elem_emb, brand_emb, item_emb, W_msg, b_msg, W_ih, W_hh, b_ih, b_hh, W_att, b_att, W_score, b_score, W_fc, b_fc):
    raise NotImplementedError("write your pallas kernel here")


# speedup vs baseline: 1.0033x; 1.0033x over previous
import jax
import jax.numpy as jnp
from jax.experimental import pallas as pl

_B = 512
_H = 100
_HP = 128
_NI_BLK = 512


def _scores_body(sess_ref, w_ref, b_ref, out_ref):
    out_ref[...] = (
        jnp.dot(sess_ref[...], w_ref[...], preferred_element_type=jnp.float32)
        + b_ref[...][None, :]
    )


def _scores_matmul(sess_pad, w_fc_t_pad, b_fc):
    B, HP = sess_pad.shape
    NI = b_fc.shape[0]
    grid = (pl.cdiv(NI, _NI_BLK),)
    return pl.pallas_call(
        _scores_body,
        grid=grid,
        in_specs=[
            pl.BlockSpec((B, HP), lambda i: (0, 0)),
            pl.BlockSpec((HP, _NI_BLK), lambda i: (0, i)),
            pl.BlockSpec((_NI_BLK,), lambda i: (i,)),
        ],
        out_specs=pl.BlockSpec((B, _NI_BLK), lambda i: (0, i)),
        out_shape=jax.ShapeDtypeStruct((B, NI), jnp.float32),
    )(sess_pad, w_fc_t_pad, b_fc)


def _gru(x, h, W_ih, W_hh, b_ih, b_hh):
    gi = x @ W_ih.T + b_ih
    gh = h @ W_hh.T + b_hh
    i_r, i_z, i_n = jnp.split(gi, 3, axis=1)
    h_r, h_z, h_n = jnp.split(gh, 3, axis=1)
    r = jax.nn.sigmoid(i_r + h_r)
    z = jax.nn.sigmoid(i_z + h_z)
    n = jnp.tanh(i_n + r * h_n)
    return (1.0 - z) * n + z * h


def kernel(price_tensor, category, sub_category, element, brand, product_id_remapped,
           edge_index, batch, cat_emb, sub_emb, elem_emb, brand_emb, item_emb,
           W_msg, b_msg, W_ih, W_hh, b_ih, b_hh, W_att, b_att, W_score, b_score,
           W_fc, b_fc):
    N = price_tensor.shape[0]
    B = _B
    emb = jnp.concatenate([
        cat_emb[category], sub_emb[sub_category], elem_emb[element],
        brand_emb[brand], item_emb[product_id_remapped]
    ], axis=1)
    x = jnp.concatenate([price_tensor, emb], axis=1)
    node = x @ W_msg.T + b_msg
    src = edge_index[0]
    dst = edge_index[1]
    msg_sum = jax.ops.segment_sum(node[src], dst, num_segments=N)
    cnt = jax.ops.segment_sum(jnp.ones((dst.shape[0],), dtype=jnp.float32), dst,
                              num_segments=N)
    messages = msg_sum / jnp.maximum(cnt, 1.0)[:, None]
    node = _gru(messages, node, W_ih, W_hh, b_ih, b_hh)
    last_idx = jax.ops.segment_max(jnp.arange(N, dtype=jnp.int32), batch,
                                   num_segments=B)
    last_idx = jnp.clip(last_idx, 0, N - 1)
    item_lt = node @ W_att.T + b_att
    last_lt = item_lt[last_idx]
    att = jax.nn.sigmoid(item_lt + last_lt[batch]) @ W_score.T + b_score
    smax = jax.ops.segment_max(att, batch, num_segments=B)
    w = jnp.exp(att - smax[batch])
    wsum = jax.ops.segment_sum(w, batch, num_segments=B)
    w = w / (wsum[batch] + 1e-16)
    sess = jax.ops.segment_sum(w * item_lt, batch, num_segments=B)

    sess_pad = jnp.pad(sess, ((0, 0), (0, _HP - _H)))
    w_fc_t_pad = jnp.pad(W_fc.T, ((0, _HP - _H), (0, 0)))
    return _scores_matmul(sess_pad, w_fc_t_pad, b_fc)


# speedup vs baseline: 2.1075x; 2.1007x over previous
import functools

import jax
import jax.numpy as jnp
from jax import lax
from jax.experimental import pallas as pl
from jax.experimental.pallas import tpu as pltpu
from jax.experimental.pallas import tpu_sc as plsc

_N = 50000
_E = 800000
_B = 512
_H = 100
_HP = 128
_NI_BLK = 512

_EPAD = 802816
_EROWS = 6272
_RPT = 392
_EB = 8
_NP = 50048
_ZROW = _NP // 16


def _edge_body(n0h, n1h, n2h, n3h, n4h, n5h, n6h, srch, dsth,
               o0, o1, o2, o3, o4, o5, o6,
               idx_s, idx_d, stage, zbuf, acc, gsem, ssem):
    c = lax.axis_index("c")
    s = lax.axis_index("s")

    @pl.loop(0, 136)
    def _z(r):
        zbuf[r, pl.ds(0, 16)] = jnp.zeros((16,), jnp.float32)

    chunks = [(n0h, o0), (n1h, o1), (n2h, o2), (n3h, o3), (n4h, o4),
              (n5h, o5), (n6h, o6)]
    for fc in range(7):
        nh, oh = chunks[fc]

        @pl.when(c == fc // 4)
        def _pass(nh=nh, oh=oh):
            @pl.loop(0, 23)
            def _zero(i):
                pltpu.sync_copy(zbuf, acc.at[pl.ds(s * _ZROW + i * 136, 136)])

            plsc.subcore_barrier()

            @pl.loop(0, _RPT // _EB)
            def _outer(it):
                r0 = s * _RPT + it * _EB
                pltpu.sync_copy(srch.at[pl.ds(r0, _EB)], idx_s)
                pltpu.sync_copy(dsth.at[pl.ds(r0, _EB)], idx_d)
                gh = [pltpu.async_copy(nh.at[idx_s.at[j]], stage.at[j], gsem)
                      for j in range(_EB)]
                for h in gh:
                    h.wait()
                sh = [pltpu.async_copy(stage.at[j], acc.at[idx_d.at[j]], ssem,
                                       add=True)
                      for j in range(_EB)]
                for h in sh:
                    h.wait()

            plsc.subcore_barrier()
            pltpu.sync_copy(acc.at[pl.ds(s * _ZROW, _ZROW)],
                            oh.at[pl.ds(s * _ZROW, _ZROW)])


def _edge_msg(nchunks, src2d, dst2d):
    mesh = plsc.VectorSubcoreMesh(core_axis_name="c", subcore_axis_name="s")
    outs = [jax.ShapeDtypeStruct((_NP, 16), jnp.float32) for _ in range(7)]
    kern = functools.partial(
        pl.kernel,
        out_type=outs,
        mesh=mesh,
        scratch_types=[
            pltpu.VMEM((_EB, 128), jnp.int32),
            pltpu.VMEM((_EB, 128), jnp.int32),
            pltpu.VMEM((_EB, 128, 16), jnp.float32),
            pltpu.VMEM((136, 16), jnp.float32),
            pltpu.VMEM_SHARED((_NP, 16), jnp.float32),
            pltpu.SemaphoreType.DMA,
            pltpu.SemaphoreType.DMA,
        ],
        compiler_params=pltpu.CompilerParams(use_tc_tiling_on_sc=False),
    )(_edge_body)
    return kern(*nchunks, src2d, dst2d)


def _scores_body(sess_ref, w_ref, b_ref, out_ref):
    out_ref[...] = (
        jnp.dot(sess_ref[...], w_ref[...], preferred_element_type=jnp.float32)
        + b_ref[...][None, :]
    )


def _scores_matmul(sess_pad, w_fc_t_pad, b_fc):
    B, HP = sess_pad.shape
    NI = b_fc.shape[0]
    grid = (pl.cdiv(NI, _NI_BLK),)
    return pl.pallas_call(
        _scores_body,
        grid=grid,
        in_specs=[
            pl.BlockSpec((B, HP), lambda i: (0, 0)),
            pl.BlockSpec((HP, _NI_BLK), lambda i: (0, i)),
            pl.BlockSpec((_NI_BLK,), lambda i: (i,)),
        ],
        out_specs=pl.BlockSpec((B, _NI_BLK), lambda i: (0, i)),
        out_shape=jax.ShapeDtypeStruct((B, NI), jnp.float32),
    )(sess_pad, w_fc_t_pad, b_fc)


def _gru(x, h, W_ih, W_hh, b_ih, b_hh):
    gi = x @ W_ih.T + b_ih
    gh = h @ W_hh.T + b_hh
    i_r, i_z, i_n = jnp.split(gi, 3, axis=1)
    h_r, h_z, h_n = jnp.split(gh, 3, axis=1)
    r = jax.nn.sigmoid(i_r + h_r)
    z = jax.nn.sigmoid(i_z + h_z)
    n = jnp.tanh(i_n + r * h_n)
    return (1.0 - z) * n + z * h


def kernel(price_tensor, category, sub_category, element, brand, product_id_remapped,
           edge_index, batch, cat_emb, sub_emb, elem_emb, brand_emb, item_emb,
           W_msg, b_msg, W_ih, W_hh, b_ih, b_hh, W_att, b_att, W_score, b_score,
           W_fc, b_fc):
    N = _N
    B = _B
    emb = jnp.concatenate([
        cat_emb[category], sub_emb[sub_category], elem_emb[element],
        brand_emb[brand], item_emb[product_id_remapped]
    ], axis=1)
    x = jnp.concatenate([price_tensor, emb], axis=1)
    node = x @ W_msg.T + b_msg

    src = edge_index[0]
    dst = edge_index[1]
    node_ext = jnp.concatenate(
        [node, jnp.ones((N, 1), jnp.float32), jnp.zeros((N, 11), jnp.float32)],
        axis=1)
    node_ext = jnp.concatenate([node_ext, jnp.zeros((_NP - N, 112), jnp.float32)],
                               axis=0)
    nchunks = [node_ext[:, 16 * i:16 * (i + 1)] for i in range(7)]
    src_p = jnp.concatenate(
        [src, jnp.full((_EPAD - _E,), N, jnp.int32)]).reshape(_EROWS, 128)
    dst_p = jnp.concatenate(
        [dst, jnp.zeros((_EPAD - _E,), jnp.int32)]).reshape(_EROWS, 128)

    mchunks = _edge_msg(nchunks, src_p, dst_p)
    msg_full = jnp.concatenate(mchunks, axis=1)[:N]
    cnt = msg_full[:, 100]
    messages = msg_full[:, :100] / jnp.maximum(cnt, 1.0)[:, None]

    node = _gru(messages, node, W_ih, W_hh, b_ih, b_hh)
    last_idx = jax.ops.segment_max(jnp.arange(N, dtype=jnp.int32), batch,
                                   num_segments=B)
    last_idx = jnp.clip(last_idx, 0, N - 1)
    item_lt = node @ W_att.T + b_att
    last_lt = item_lt[last_idx]
    att = jax.nn.sigmoid(item_lt + last_lt[batch]) @ W_score.T + b_score
    smax = jax.ops.segment_max(att, batch, num_segments=B)
    w = jnp.exp(att - smax[batch])
    wsum = jax.ops.segment_sum(w, batch, num_segments=B)
    w = w / (wsum[batch] + 1e-16)
    sess = jax.ops.segment_sum(w * item_lt, batch, num_segments=B)

    sess_pad = jnp.pad(sess, ((0, 0), (0, _HP - _H)))
    w_fc_t_pad = jnp.pad(W_fc.T, ((0, _HP - _H), (0, 0)))
    return _scores_matmul(sess_pad, w_fc_t_pad, b_fc)


# speedup vs baseline: 3.2312x; 1.5332x over previous
import functools

import jax
import jax.numpy as jnp
from jax import lax
from jax.experimental import pallas as pl
from jax.experimental.pallas import tpu as pltpu
from jax.experimental.pallas import tpu_sc as plsc

_N = 50000
_E = 800000
_B = 512
_H = 100
_W = 112
_NP = 50048
_NPB = 50176
_EPAD = 802816
_EROWS = 6272
_RPT = 392
_EB = 8
_ZROW = _NP // 16
_TBLK = 1024
_TG = 49
_NI_BLK = 512
_IDXROWS = 391


_IDXP = 416
_GOUT = _IDXP * 128


def _emb_body(t0, t1, t2, t3, t4, i0, i1, i2, i3, i4,
              o0, o1, o2, o3, o4, idxb, stage, sem):
    c = lax.axis_index("c")
    s = lax.axis_index("s")
    w = c * 16 + s
    r0 = w * 13
    for t in range(5):
        tab = (t0, t1, t2, t3, t4)[t]
        idx2 = (i0, i1, i2, i3, i4)[t]
        out = (o0, o1, o2, o3, o4)[t]
        pltpu.sync_copy(idx2.at[pl.ds(r0, 13)], idxb)
        hs = [pltpu.async_copy(tab.at[idxb.at[j]], stage.at[j], sem)
              for j in range(13)]
        for h in hs:
            h.wait()
        for j in range(13):
            pltpu.sync_copy(stage.at[j], out.at[pl.ds((r0 + j) * 128, 128)])


def _emb_gather(tables, idxs):
    mesh = plsc.VectorSubcoreMesh(core_axis_name="c", subcore_axis_name="s")
    outs = [jax.ShapeDtypeStruct((_GOUT, 32), jnp.float32) for _ in range(5)]
    kern = functools.partial(
        pl.kernel,
        out_type=outs,
        mesh=mesh,
        scratch_types=[
            pltpu.VMEM((13, 128), jnp.int32),
            pltpu.VMEM((13, 128, 32), jnp.float32),
            pltpu.SemaphoreType.DMA,
        ],
        compiler_params=pltpu.CompilerParams(use_tc_tiling_on_sc=False),
    )(_emb_body)
    return kern(*tables, *idxs)


def _proj_body(price, g0, g1, g2, g3, g4, wg, wp, brow, *outs):
    xg = jnp.concatenate([g0[...], g1[...], g2[...], g3[...], g4[...]], axis=1)
    node = (jnp.dot(xg, wg[...], preferred_element_type=jnp.float32)
            + price[...] * wp[...] + brow[...])
    for t in range(7):
        outs[t][...] = node[:, 16 * t:16 * (t + 1)]


def _proj(price_pad, gs, wg, wp, brow):
    outs = [jax.ShapeDtypeStruct((_NP, 16), jnp.float32) for _ in range(7)]
    return pl.pallas_call(
        _proj_body,
        grid=(_TG,),
        in_specs=[
            pl.BlockSpec((_TBLK, 1), lambda i: (i, 0)),
            *[pl.BlockSpec((_TBLK, 32), lambda i: (i, 0)) for _ in range(5)],
            pl.BlockSpec((160, _W), lambda i: (0, 0)),
            pl.BlockSpec((1, _W), lambda i: (0, 0)),
            pl.BlockSpec((1, _W), lambda i: (0, 0)),
        ],
        out_specs=[pl.BlockSpec((_TBLK, 16), lambda i: (i, 0))
                   for _ in range(7)],
        out_shape=outs,
    )(price_pad, *gs, wg, wp, brow)


def _edge_body(n0h, n1h, n2h, n3h, n4h, n5h, n6h, srch, dsth,
               o0, o1, o2, o3, o4, o5, o6,
               idx_s, idx_d, stage, zbuf, acc, gsem, ssem):
    c = lax.axis_index("c")
    s = lax.axis_index("s")

    @pl.loop(0, 136)
    def _z(r):
        zbuf[r, pl.ds(0, 16)] = jnp.zeros((16,), jnp.float32)

    chunks = [(n0h, o0), (n1h, o1), (n2h, o2), (n3h, o3), (n4h, o4),
              (n5h, o5), (n6h, o6)]
    for fc in range(7):
        nh, oh = chunks[fc]

        @pl.when(c == fc // 4)
        def _pass(nh=nh, oh=oh):
            @pl.loop(0, 23)
            def _zero(i):
                pltpu.sync_copy(zbuf, acc.at[pl.ds(s * _ZROW + i * 136, 136)])

            plsc.subcore_barrier()

            @pl.loop(0, _RPT // _EB)
            def _outer(it):
                r0 = s * _RPT + it * _EB
                pltpu.sync_copy(srch.at[pl.ds(r0, _EB)], idx_s)
                pltpu.sync_copy(dsth.at[pl.ds(r0, _EB)], idx_d)
                gh = [pltpu.async_copy(nh.at[idx_s.at[j]], stage.at[j], gsem)
                      for j in range(_EB)]
                for h in gh:
                    h.wait()
                sh = [pltpu.async_copy(stage.at[j], acc.at[idx_d.at[j]], ssem,
                                       add=True)
                      for j in range(_EB)]
                for h in sh:
                    h.wait()

            plsc.subcore_barrier()
            pltpu.sync_copy(acc.at[pl.ds(s * _ZROW, _ZROW)],
                            oh.at[pl.ds(s * _ZROW, _ZROW)])


def _edge_msg(nchunks, src2d, dst2d):
    mesh = plsc.VectorSubcoreMesh(core_axis_name="c", subcore_axis_name="s")
    outs = [jax.ShapeDtypeStruct((_NP, 16), jnp.float32) for _ in range(7)]
    kern = functools.partial(
        pl.kernel,
        out_type=outs,
        mesh=mesh,
        scratch_types=[
            pltpu.VMEM((_EB, 128), jnp.int32),
            pltpu.VMEM((_EB, 128), jnp.int32),
            pltpu.VMEM((_EB, 128, 16), jnp.float32),
            pltpu.VMEM((136, 16), jnp.float32),
            pltpu.VMEM_SHARED((_NP, 16), jnp.float32),
            pltpu.SemaphoreType.DMA,
            pltpu.SemaphoreType.DMA,
        ],
        compiler_params=pltpu.CompilerParams(use_tc_tiling_on_sc=False),
    )(_edge_body)
    return kern(*nchunks, src2d, dst2d)


def _gru_body(m0, m1, m2, m3, m4, m5, m6, n0, n1, n2, n3, n4, n5, n6,
              wih, whh, bi, bh, watt, batt, ilt_out):
    msgf = jnp.concatenate([m[...] for m in (m0, m1, m2, m3, m4, m5, m6)],
                           axis=1)
    h = jnp.concatenate([n[...] for n in (n0, n1, n2, n3, n4, n5, n6)],
                        axis=1)
    cnt = jnp.maximum(msgf[:, 100:101], 1.0)
    x = msgf / cnt
    gi = jnp.dot(x, wih[...], preferred_element_type=jnp.float32) + bi[...]
    gh = jnp.dot(h, whh[...], preferred_element_type=jnp.float32) + bh[...]
    r = jax.nn.sigmoid(gi[:, 0:_W] + gh[:, 0:_W])
    z = jax.nn.sigmoid(gi[:, _W:2 * _W] + gh[:, _W:2 * _W])
    n = jnp.tanh(gi[:, 2 * _W:3 * _W] + r * gh[:, 2 * _W:3 * _W])
    hn = (1.0 - z) * n + z * h
    ilt_out[...] = (jnp.dot(hn, watt[...], preferred_element_type=jnp.float32)
                    + batt[...])


def _gru_att(mchunks, nchunks, wih, whh, bi, bh, watt, batt):
    return pl.pallas_call(
        _gru_body,
        grid=(_TG,),
        in_specs=[
            *[pl.BlockSpec((_TBLK, 16), lambda i: (i, 0)) for _ in range(14)],
            pl.BlockSpec((_W, 3 * _W), lambda i: (0, 0)),
            pl.BlockSpec((_W, 3 * _W), lambda i: (0, 0)),
            pl.BlockSpec((1, 3 * _W), lambda i: (0, 0)),
            pl.BlockSpec((1, 3 * _W), lambda i: (0, 0)),
            pl.BlockSpec((_W, _W), lambda i: (0, 0)),
            pl.BlockSpec((1, _W), lambda i: (0, 0)),
        ],
        out_specs=pl.BlockSpec((_TBLK, _W), lambda i: (i, 0)),
        out_shape=jax.ShapeDtypeStruct((_NP, _W), jnp.float32),
    )(*mchunks, *nchunks, wih, whh, bi, bh, watt, batt)


def _lastlt_body(ilt, b3, bn3, out, acc):
    i = pl.program_id(0)
    bvec = b3[0, 0, :]
    nvec = bn3[0, 0, :]
    oh = bvec[:, None] == lax.broadcasted_iota(jnp.int32, (_TBLK, _B), 1)
    m = bvec[:, None] != nvec[:, None]
    ohm = jnp.logical_and(oh, m).astype(jnp.float32)
    part = lax.dot_general(ohm, ilt[...], (((0,), (0,)), ((), ())),
                           preferred_element_type=jnp.float32)

    @pl.when(i == 0)
    def _():
        acc[...] = jnp.zeros((_B, _W), jnp.float32)

    acc[...] += part

    @pl.when(i == _TG - 1)
    def _():
        out[...] = acc[...]


def _last_gather(batch3, bnext3, item_lt):
    return pl.pallas_call(
        _lastlt_body,
        grid=(_TG,),
        in_specs=[
            pl.BlockSpec((_TBLK, _W), lambda i: (i, 0)),
            pl.BlockSpec((1, 1, _TBLK), lambda i: (i, 0, 0)),
            pl.BlockSpec((1, 1, _TBLK), lambda i: (i, 0, 0)),
        ],
        out_specs=pl.BlockSpec((_B, _W), lambda i: (0, 0)),
        out_shape=jax.ShapeDtypeStruct((_B, _W), jnp.float32),
        scratch_shapes=[pltpu.VMEM((_B, _W), jnp.float32)],
    )(item_lt, batch3, bnext3)


def _att_body(ilt, b3, llt, wsc, bsc, att_out, smax_out, acc):
    i = pl.program_id(0)
    bvec = b3[0, 0, :]
    oh = bvec[:, None] == lax.broadcasted_iota(jnp.int32, (_TBLK, _B), 1)
    ohf = oh.astype(jnp.float32)
    expand = jnp.dot(ohf, llt[...], preferred_element_type=jnp.float32)
    sg = jax.nn.sigmoid(ilt[...] + expand)
    att = jnp.sum(sg * wsc[...], axis=1, keepdims=True) + bsc[0, 0]
    att_out[...] = att
    rows = i * _TBLK + lax.broadcasted_iota(jnp.int32, (_TBLK, 1), 0)
    valid = rows < _N
    attm = jnp.where(jnp.logical_and(oh, valid), att, -1e30)
    part = jnp.max(attm, axis=0, keepdims=True)

    @pl.when(i == 0)
    def _():
        acc[...] = jnp.full((1, _B), -1e30, jnp.float32)

    acc[...] = jnp.maximum(acc[...], part)

    @pl.when(i == _TG - 1)
    def _():
        smax_out[...] = acc[...]


def _att_smax(item_lt, batch3, last_lt, wsc, bsc):
    return pl.pallas_call(
        _att_body,
        grid=(_TG,),
        in_specs=[
            pl.BlockSpec((_TBLK, _W), lambda i: (i, 0)),
            pl.BlockSpec((1, 1, _TBLK), lambda i: (i, 0, 0)),
            pl.BlockSpec((_B, _W), lambda i: (0, 0)),
            pl.BlockSpec((1, _W), lambda i: (0, 0)),
            pl.BlockSpec((1, 1), lambda i: (0, 0)),
        ],
        out_specs=[
            pl.BlockSpec((_TBLK, 1), lambda i: (i, 0)),
            pl.BlockSpec((1, _B), lambda i: (0, 0)),
        ],
        out_shape=[
            jax.ShapeDtypeStruct((_NPB, 1), jnp.float32),
            jax.ShapeDtypeStruct((1, _B), jnp.float32),
        ],
        scratch_shapes=[pltpu.VMEM((1, _B), jnp.float32)],
    )(item_lt, batch3, last_lt, wsc, bsc)


def _pool_body(att, b3, ilt, smax, wsum_out, sessu_out, accw, accs):
    i = pl.program_id(0)
    bvec = b3[0, 0, :]
    oh = bvec[:, None] == lax.broadcasted_iota(jnp.int32, (_TBLK, _B), 1)
    ohf = oh.astype(jnp.float32)
    rows = i * _TBLK + lax.broadcasted_iota(jnp.int32, (_TBLK, 1), 0)
    valid = rows < _N
    sm_exp = jnp.sum(ohf * smax[...], axis=1, keepdims=True)
    e = jnp.where(valid, jnp.exp(att[...] - sm_exp), 0.0)
    pw = lax.dot_general(ohf, e, (((0,), (0,)), ((), ())),
                         preferred_element_type=jnp.float32)
    ps = lax.dot_general(ohf, e * ilt[...], (((0,), (0,)), ((), ())),
                         preferred_element_type=jnp.float32)

    @pl.when(i == 0)
    def _():
        accw[...] = jnp.zeros((_B, 1), jnp.float32)
        accs[...] = jnp.zeros((_B, _W), jnp.float32)

    accw[...] += pw
    accs[...] += ps

    @pl.when(i == _TG - 1)
    def _():
        wsum_out[...] = accw[...]
        sessu_out[...] = accs[...]


def _pool(att, batch3, item_lt, smax):
    return pl.pallas_call(
        _pool_body,
        grid=(_TG,),
        in_specs=[
            pl.BlockSpec((_TBLK, 1), lambda i: (i, 0)),
            pl.BlockSpec((1, 1, _TBLK), lambda i: (i, 0, 0)),
            pl.BlockSpec((_TBLK, _W), lambda i: (i, 0)),
            pl.BlockSpec((1, _B), lambda i: (0, 0)),
        ],
        out_specs=[
            pl.BlockSpec((_B, 1), lambda i: (0, 0)),
            pl.BlockSpec((_B, _W), lambda i: (0, 0)),
        ],
        out_shape=[
            jax.ShapeDtypeStruct((_B, 1), jnp.float32),
            jax.ShapeDtypeStruct((_B, _W), jnp.float32),
        ],
        scratch_shapes=[
            pltpu.VMEM((_B, 1), jnp.float32),
            pltpu.VMEM((_B, _W), jnp.float32),
        ],
    )(att, batch3, item_lt, smax)


def _scores_body(sessu, wsum, wfc, bfc, out):
    sess = sessu[...] / (wsum[...] + 1e-16)
    out[...] = (lax.dot_general(sess[:, :_H], wfc[...],
                                (((1,), (1,)), ((), ())),
                                preferred_element_type=jnp.float32)
                + bfc[...])


def _scores(sessu, wsum, w_fc, b_fc2):
    NI = w_fc.shape[0]
    return pl.pallas_call(
        _scores_body,
        grid=(pl.cdiv(NI, _NI_BLK),),
        in_specs=[
            pl.BlockSpec((_B, _W), lambda i: (0, 0)),
            pl.BlockSpec((_B, 1), lambda i: (0, 0)),
            pl.BlockSpec((_NI_BLK, _H), lambda i: (i, 0)),
            pl.BlockSpec((1, _NI_BLK), lambda i: (0, i)),
        ],
        out_specs=pl.BlockSpec((_B, _NI_BLK), lambda i: (0, i)),
        out_shape=jax.ShapeDtypeStruct((_B, NI), jnp.float32),
    )(sessu, wsum, w_fc, b_fc2)


def kernel(price_tensor, category, sub_category, element, brand, product_id_remapped,
           edge_index, batch, cat_emb, sub_emb, elem_emb, brand_emb, item_emb,
           W_msg, b_msg, W_ih, W_hh, b_ih, b_hh, W_att, b_att, W_score, b_score,
           W_fc, b_fc):
    N, B, H, W = _N, _B, _H, _W
    f32 = jnp.float32

    tables = [jnp.pad(t, ((0, 0), (0, 32 - 25)))
              for t in (cat_emb, sub_emb, elem_emb, brand_emb, item_emb)]
    idxs = [jnp.pad(ix, (0, _GOUT - N)).reshape(_IDXP, 128)
            for ix in (category, sub_category, element, brand,
                       product_id_remapped)]
    gs = _emb_gather(tables, idxs)

    wg = jnp.zeros((160, W), f32)
    for t in range(5):
        wg = wg.at[32 * t:32 * t + 25, :H].set(W_msg[:, 1 + 25 * t:26 + 25 * t].T)
    wp = jnp.pad(W_msg[:, 0], (0, W - H)).reshape(1, W)
    brow = jnp.concatenate([b_msg, jnp.ones((1,), f32),
                            jnp.zeros((W - H - 1,), f32)]).reshape(1, W)
    price_pad = jnp.pad(price_tensor, ((0, _NP - N), (0, 0)))
    nchunks = _proj(price_pad, gs, wg, wp, brow)

    src = edge_index[0]
    dst = edge_index[1]
    src_p = jnp.concatenate(
        [src, jnp.full((_EPAD - _E,), N, jnp.int32)]).reshape(_EROWS, 128)
    dst_p = jnp.concatenate(
        [dst, jnp.full((_EPAD - _E,), N, jnp.int32)]).reshape(_EROWS, 128)
    mchunks = _edge_msg(nchunks, src_p, dst_p)

    wih = jnp.zeros((W, 3 * W), f32)
    whh = jnp.zeros((W, 3 * W), f32)
    bi = jnp.zeros((1, 3 * W), f32)
    bh = jnp.zeros((1, 3 * W), f32)
    for g in range(3):
        wih = wih.at[:H, W * g:W * g + H].set(W_ih[H * g:H * (g + 1), :].T)
        whh = whh.at[:H, W * g:W * g + H].set(W_hh[H * g:H * (g + 1), :].T)
        bi = bi.at[0, W * g:W * g + H].set(b_ih[H * g:H * (g + 1)])
        bh = bh.at[0, W * g:W * g + H].set(b_hh[H * g:H * (g + 1)])
    watt = jnp.pad(W_att.T, ((0, W - H), (0, W - H)))
    batt = jnp.pad(b_att, (0, W - H)).reshape(1, W)
    item_lt = _gru_att(mchunks, nchunks, wih, whh, bi, bh, watt, batt)

    bflat = jnp.concatenate([batch.astype(jnp.int32),
                             jnp.full((_NPB - N,), B, jnp.int32)])
    bshift = jnp.concatenate([bflat[1:], jnp.full((1,), B, jnp.int32)])
    batch3 = bflat.reshape(_TG, 1, _TBLK)
    bnext3 = bshift.reshape(_TG, 1, _TBLK)
    last_lt = _last_gather(batch3, bnext3, item_lt)

    wsc = jnp.pad(W_score[0], (0, W - H)).reshape(1, W)
    bsc = b_score.reshape(1, 1)
    att, smax = _att_smax(item_lt, batch3, last_lt, wsc, bsc)
    wsum, sessu = _pool(att, batch3, item_lt, smax)

    b_fc2 = b_fc.reshape(1, -1)
    return _scores(sessu, wsum, W_fc, b_fc2)


# speedup vs baseline: 3.4099x; 1.0553x over previous
import functools

import jax
import jax.numpy as jnp
from jax import lax
from jax.experimental import pallas as pl
from jax.experimental.pallas import tpu as pltpu
from jax.experimental.pallas import tpu_sc as plsc

_N = 50000
_E = 800000
_B = 512
_H = 100
_W = 112
_NP = 50048
_NPB = 50176
_EPAD = 802816
_EROWS = 6272
_RPT = 392
_EB = 14
_ZROW = _NP // 16
_TBLK = 1024
_TG = 49
_NI_BLK = 512
_IDXROWS = 391


_IDXP = 416
_GOUT = _IDXP * 128


def _emb_body(t0, t1, t2, t3, t4, i0, i1, i2, i3, i4,
              o0, o1, o2, o3, o4, idxb, stage, sem):
    c = lax.axis_index("c")
    s = lax.axis_index("s")
    w = c * 16 + s
    r0 = w * 13
    for t in range(5):
        tab = (t0, t1, t2, t3, t4)[t]
        idx2 = (i0, i1, i2, i3, i4)[t]
        out = (o0, o1, o2, o3, o4)[t]
        pltpu.sync_copy(idx2.at[pl.ds(r0, 13)], idxb)
        hs = [pltpu.async_copy(tab.at[idxb.at[j]], stage.at[j], sem)
              for j in range(13)]
        for h in hs:
            h.wait()
        for j in range(13):
            pltpu.sync_copy(stage.at[j], out.at[pl.ds((r0 + j) * 128, 128)])


def _emb_gather(tables, idxs):
    mesh = plsc.VectorSubcoreMesh(core_axis_name="c", subcore_axis_name="s")
    outs = [jax.ShapeDtypeStruct((_GOUT, 32), jnp.float32) for _ in range(5)]
    kern = functools.partial(
        pl.kernel,
        out_type=outs,
        mesh=mesh,
        scratch_types=[
            pltpu.VMEM((13, 128), jnp.int32),
            pltpu.VMEM((13, 128, 32), jnp.float32),
            pltpu.SemaphoreType.DMA,
        ],
        compiler_params=pltpu.CompilerParams(use_tc_tiling_on_sc=False),
    )(_emb_body)
    return kern(*tables, *idxs)


def _proj_body(price, g0, g1, g2, g3, g4, wg, wp, brow, *outs):
    xg = jnp.concatenate([g0[...], g1[...], g2[...], g3[...], g4[...]], axis=1)
    node = (jnp.dot(xg, wg[...], preferred_element_type=jnp.float32)
            + price[...] * wp[...] + brow[...])
    for t in range(7):
        outs[t][...] = node[:, 16 * t:16 * (t + 1)]


def _proj(price_pad, gs, wg, wp, brow):
    outs = [jax.ShapeDtypeStruct((_NP, 16), jnp.float32) for _ in range(7)]
    return pl.pallas_call(
        _proj_body,
        grid=(_TG,),
        in_specs=[
            pl.BlockSpec((_TBLK, 1), lambda i: (i, 0)),
            *[pl.BlockSpec((_TBLK, 32), lambda i: (i, 0)) for _ in range(5)],
            pl.BlockSpec((160, _W), lambda i: (0, 0)),
            pl.BlockSpec((1, _W), lambda i: (0, 0)),
            pl.BlockSpec((1, _W), lambda i: (0, 0)),
        ],
        out_specs=[pl.BlockSpec((_TBLK, 16), lambda i: (i, 0))
                   for _ in range(7)],
        out_shape=outs,
    )(price_pad, *gs, wg, wp, brow)


def _edge_body(n0h, n1h, n2h, n3h, n4h, n5h, n6h, srch, dsth,
               o0, o1, o2, o3, o4, o5, o6,
               idx_s, idx_d, stage, zbuf, acc, gsem, ssem):
    c = lax.axis_index("c")
    s = lax.axis_index("s")

    @pl.loop(0, 136)
    def _z(r):
        zbuf[r, pl.ds(0, 16)] = jnp.zeros((16,), jnp.float32)

    chunks = [(n0h, o0), (n1h, o1), (n2h, o2), (n3h, o3), (n4h, o4),
              (n5h, o5), (n6h, o6)]
    for fc in range(7):
        nh, oh = chunks[fc]

        @pl.when(c == fc // 4)
        def _pass(nh=nh, oh=oh):
            @pl.loop(0, 23)
            def _zero(i):
                pltpu.sync_copy(zbuf, acc.at[pl.ds(s * _ZROW + i * 136, 136)])

            plsc.subcore_barrier()

            @pl.loop(0, _RPT // _EB)
            def _outer(it):
                r0 = s * _RPT + it * _EB
                pltpu.sync_copy(srch.at[pl.ds(r0, _EB)], idx_s)
                pltpu.sync_copy(dsth.at[pl.ds(r0, _EB)], idx_d)
                gh = [pltpu.async_copy(nh.at[idx_s.at[j]], stage.at[j], gsem)
                      for j in range(_EB)]
                for h in gh:
                    h.wait()
                sh = [pltpu.async_copy(stage.at[j], acc.at[idx_d.at[j]], ssem,
                                       add=True)
                      for j in range(_EB)]
                for h in sh:
                    h.wait()

            plsc.subcore_barrier()
            pltpu.sync_copy(acc.at[pl.ds(s * _ZROW, _ZROW)],
                            oh.at[pl.ds(s * _ZROW, _ZROW)])


def _edge_msg(nchunks, src2d, dst2d):
    mesh = plsc.VectorSubcoreMesh(core_axis_name="c", subcore_axis_name="s")
    outs = [jax.ShapeDtypeStruct((_NP, 16), jnp.float32) for _ in range(7)]
    kern = functools.partial(
        pl.kernel,
        out_type=outs,
        mesh=mesh,
        scratch_types=[
            pltpu.VMEM((_EB, 128), jnp.int32),
            pltpu.VMEM((_EB, 128), jnp.int32),
            pltpu.VMEM((_EB, 128, 16), jnp.float32),
            pltpu.VMEM((136, 16), jnp.float32),
            pltpu.VMEM_SHARED((_NP, 16), jnp.float32),
            pltpu.SemaphoreType.DMA,
            pltpu.SemaphoreType.DMA,
        ],
        compiler_params=pltpu.CompilerParams(use_tc_tiling_on_sc=False),
    )(_edge_body)
    return kern(*nchunks, src2d, dst2d)


def _gru_body(m0, m1, m2, m3, m4, m5, m6, n0, n1, n2, n3, n4, n5, n6,
              wih, whh, bi, bh, watt, batt, ilt_out):
    msgf = jnp.concatenate([m[...] for m in (m0, m1, m2, m3, m4, m5, m6)],
                           axis=1)
    h = jnp.concatenate([n[...] for n in (n0, n1, n2, n3, n4, n5, n6)],
                        axis=1)
    cnt = jnp.maximum(msgf[:, 100:101], 1.0)
    x = msgf / cnt
    gi = jnp.dot(x, wih[...], preferred_element_type=jnp.float32) + bi[...]
    gh = jnp.dot(h, whh[...], preferred_element_type=jnp.float32) + bh[...]
    r = jax.nn.sigmoid(gi[:, 0:_W] + gh[:, 0:_W])
    z = jax.nn.sigmoid(gi[:, _W:2 * _W] + gh[:, _W:2 * _W])
    n = jnp.tanh(gi[:, 2 * _W:3 * _W] + r * gh[:, 2 * _W:3 * _W])
    hn = (1.0 - z) * n + z * h
    ilt_out[...] = (jnp.dot(hn, watt[...], preferred_element_type=jnp.float32)
                    + batt[...])


def _gru_att(mchunks, nchunks, wih, whh, bi, bh, watt, batt):
    return pl.pallas_call(
        _gru_body,
        grid=(_TG,),
        in_specs=[
            *[pl.BlockSpec((_TBLK, 16), lambda i: (i, 0)) for _ in range(14)],
            pl.BlockSpec((_W, 3 * _W), lambda i: (0, 0)),
            pl.BlockSpec((_W, 3 * _W), lambda i: (0, 0)),
            pl.BlockSpec((1, 3 * _W), lambda i: (0, 0)),
            pl.BlockSpec((1, 3 * _W), lambda i: (0, 0)),
            pl.BlockSpec((_W, _W), lambda i: (0, 0)),
            pl.BlockSpec((1, _W), lambda i: (0, 0)),
        ],
        out_specs=pl.BlockSpec((_TBLK, _W), lambda i: (i, 0)),
        out_shape=jax.ShapeDtypeStruct((_NP, _W), jnp.float32),
    )(*mchunks, *nchunks, wih, whh, bi, bh, watt, batt)


def _lastlt_body(ilt, b3, bn3, out, acc):
    i = pl.program_id(0)
    bvec = b3[0, 0, :]
    nvec = bn3[0, 0, :]
    oh = bvec[:, None] == lax.broadcasted_iota(jnp.int32, (_TBLK, _B), 1)
    m = bvec[:, None] != nvec[:, None]
    ohm = jnp.logical_and(oh, m).astype(jnp.float32)
    part = lax.dot_general(ohm, ilt[...], (((0,), (0,)), ((), ())),
                           preferred_element_type=jnp.float32)

    @pl.when(i == 0)
    def _():
        acc[...] = jnp.zeros((_B, _W), jnp.float32)

    acc[...] += part

    @pl.when(i == _TG - 1)
    def _():
        out[...] = acc[...]


def _last_gather(batch3, bnext3, item_lt):
    return pl.pallas_call(
        _lastlt_body,
        grid=(_TG,),
        in_specs=[
            pl.BlockSpec((_TBLK, _W), lambda i: (i, 0)),
            pl.BlockSpec((1, 1, _TBLK), lambda i: (i, 0, 0)),
            pl.BlockSpec((1, 1, _TBLK), lambda i: (i, 0, 0)),
        ],
        out_specs=pl.BlockSpec((_B, _W), lambda i: (0, 0)),
        out_shape=jax.ShapeDtypeStruct((_B, _W), jnp.float32),
        scratch_shapes=[pltpu.VMEM((_B, _W), jnp.float32)],
    )(item_lt, batch3, bnext3)


def _att_body(ilt, b3, llt, wsc, bsc, att_out, smax_out, acc):
    i = pl.program_id(0)
    bvec = b3[0, 0, :]
    oh = bvec[:, None] == lax.broadcasted_iota(jnp.int32, (_TBLK, _B), 1)
    ohf = oh.astype(jnp.float32)
    expand = jnp.dot(ohf, llt[...], preferred_element_type=jnp.float32)
    sg = jax.nn.sigmoid(ilt[...] + expand)
    att = jnp.sum(sg * wsc[...], axis=1, keepdims=True) + bsc[0, 0]
    att_out[...] = att
    rows = i * _TBLK + lax.broadcasted_iota(jnp.int32, (_TBLK, 1), 0)
    valid = rows < _N
    attm = jnp.where(jnp.logical_and(oh, valid), att, -1e30)
    part = jnp.max(attm, axis=0, keepdims=True)

    @pl.when(i == 0)
    def _():
        acc[...] = jnp.full((1, _B), -1e30, jnp.float32)

    acc[...] = jnp.maximum(acc[...], part)

    @pl.when(i == _TG - 1)
    def _():
        smax_out[...] = acc[...]


def _att_smax(item_lt, batch3, last_lt, wsc, bsc):
    return pl.pallas_call(
        _att_body,
        grid=(_TG,),
        in_specs=[
            pl.BlockSpec((_TBLK, _W), lambda i: (i, 0)),
            pl.BlockSpec((1, 1, _TBLK), lambda i: (i, 0, 0)),
            pl.BlockSpec((_B, _W), lambda i: (0, 0)),
            pl.BlockSpec((1, _W), lambda i: (0, 0)),
            pl.BlockSpec((1, 1), lambda i: (0, 0)),
        ],
        out_specs=[
            pl.BlockSpec((_TBLK, 1), lambda i: (i, 0)),
            pl.BlockSpec((1, _B), lambda i: (0, 0)),
        ],
        out_shape=[
            jax.ShapeDtypeStruct((_NPB, 1), jnp.float32),
            jax.ShapeDtypeStruct((1, _B), jnp.float32),
        ],
        scratch_shapes=[pltpu.VMEM((1, _B), jnp.float32)],
    )(item_lt, batch3, last_lt, wsc, bsc)


def _pool_body(att, b3, ilt, smax, wsum_out, sessu_out, accw, accs):
    i = pl.program_id(0)
    bvec = b3[0, 0, :]
    oh = bvec[:, None] == lax.broadcasted_iota(jnp.int32, (_TBLK, _B), 1)
    ohf = oh.astype(jnp.float32)
    rows = i * _TBLK + lax.broadcasted_iota(jnp.int32, (_TBLK, 1), 0)
    valid = rows < _N
    sm_exp = jnp.sum(ohf * smax[...], axis=1, keepdims=True)
    e = jnp.where(valid, jnp.exp(att[...] - sm_exp), 0.0)
    pw = lax.dot_general(ohf, e, (((0,), (0,)), ((), ())),
                         preferred_element_type=jnp.float32)
    ps = lax.dot_general(ohf, e * ilt[...], (((0,), (0,)), ((), ())),
                         preferred_element_type=jnp.float32)

    @pl.when(i == 0)
    def _():
        accw[...] = jnp.zeros((_B, 1), jnp.float32)
        accs[...] = jnp.zeros((_B, _W), jnp.float32)

    accw[...] += pw
    accs[...] += ps

    @pl.when(i == _TG - 1)
    def _():
        wsum_out[...] = accw[...]
        sessu_out[...] = accs[...]


def _pool(att, batch3, item_lt, smax):
    return pl.pallas_call(
        _pool_body,
        grid=(_TG,),
        in_specs=[
            pl.BlockSpec((_TBLK, 1), lambda i: (i, 0)),
            pl.BlockSpec((1, 1, _TBLK), lambda i: (i, 0, 0)),
            pl.BlockSpec((_TBLK, _W), lambda i: (i, 0)),
            pl.BlockSpec((1, _B), lambda i: (0, 0)),
        ],
        out_specs=[
            pl.BlockSpec((_B, 1), lambda i: (0, 0)),
            pl.BlockSpec((_B, _W), lambda i: (0, 0)),
        ],
        out_shape=[
            jax.ShapeDtypeStruct((_B, 1), jnp.float32),
            jax.ShapeDtypeStruct((_B, _W), jnp.float32),
        ],
        scratch_shapes=[
            pltpu.VMEM((_B, 1), jnp.float32),
            pltpu.VMEM((_B, _W), jnp.float32),
        ],
    )(att, batch3, item_lt, smax)


def _scores_body(sessu, wsum, wfc, bfc, out):
    sess = sessu[...] / (wsum[...] + 1e-16)
    out[...] = (lax.dot_general(sess[:, :_H], wfc[...],
                                (((1,), (1,)), ((), ())),
                                preferred_element_type=jnp.float32)
                + bfc[...])


def _scores(sessu, wsum, w_fc, b_fc2):
    NI = w_fc.shape[0]
    return pl.pallas_call(
        _scores_body,
        grid=(pl.cdiv(NI, _NI_BLK),),
        in_specs=[
            pl.BlockSpec((_B, _W), lambda i: (0, 0)),
            pl.BlockSpec((_B, 1), lambda i: (0, 0)),
            pl.BlockSpec((_NI_BLK, _H), lambda i: (i, 0)),
            pl.BlockSpec((1, _NI_BLK), lambda i: (0, i)),
        ],
        out_specs=pl.BlockSpec((_B, _NI_BLK), lambda i: (0, i)),
        out_shape=jax.ShapeDtypeStruct((_B, NI), jnp.float32),
    )(sessu, wsum, w_fc, b_fc2)


def kernel(price_tensor, category, sub_category, element, brand, product_id_remapped,
           edge_index, batch, cat_emb, sub_emb, elem_emb, brand_emb, item_emb,
           W_msg, b_msg, W_ih, W_hh, b_ih, b_hh, W_att, b_att, W_score, b_score,
           W_fc, b_fc):
    N, B, H, W = _N, _B, _H, _W
    f32 = jnp.float32

    tables = [jnp.pad(t, ((0, 0), (0, 32 - 25)))
              for t in (cat_emb, sub_emb, elem_emb, brand_emb, item_emb)]
    idxs = [jnp.pad(ix, (0, _GOUT - N)).reshape(_IDXP, 128)
            for ix in (category, sub_category, element, brand,
                       product_id_remapped)]
    gs = _emb_gather(tables, idxs)

    wg = jnp.zeros((160, W), f32)
    for t in range(5):
        wg = wg.at[32 * t:32 * t + 25, :H].set(W_msg[:, 1 + 25 * t:26 + 25 * t].T)
    wp = jnp.pad(W_msg[:, 0], (0, W - H)).reshape(1, W)
    brow = jnp.concatenate([b_msg, jnp.ones((1,), f32),
                            jnp.zeros((W - H - 1,), f32)]).reshape(1, W)
    price_pad = jnp.pad(price_tensor, ((0, _NP - N), (0, 0)))
    nchunks = _proj(price_pad, gs, wg, wp, brow)

    src = edge_index[0]
    dst = edge_index[1]
    src_p = jnp.concatenate(
        [src, jnp.full((_EPAD - _E,), N, jnp.int32)]).reshape(_EROWS, 128)
    dst_p = jnp.concatenate(
        [dst, jnp.full((_EPAD - _E,), N, jnp.int32)]).reshape(_EROWS, 128)
    mchunks = _edge_msg(nchunks, src_p, dst_p)

    wih = jnp.zeros((W, 3 * W), f32)
    whh = jnp.zeros((W, 3 * W), f32)
    bi = jnp.zeros((1, 3 * W), f32)
    bh = jnp.zeros((1, 3 * W), f32)
    for g in range(3):
        wih = wih.at[:H, W * g:W * g + H].set(W_ih[H * g:H * (g + 1), :].T)
        whh = whh.at[:H, W * g:W * g + H].set(W_hh[H * g:H * (g + 1), :].T)
        bi = bi.at[0, W * g:W * g + H].set(b_ih[H * g:H * (g + 1)])
        bh = bh.at[0, W * g:W * g + H].set(b_hh[H * g:H * (g + 1)])
    watt = jnp.pad(W_att.T, ((0, W - H), (0, W - H)))
    batt = jnp.pad(b_att, (0, W - H)).reshape(1, W)
    item_lt = _gru_att(mchunks, nchunks, wih, whh, bi, bh, watt, batt)

    bflat = jnp.concatenate([batch.astype(jnp.int32),
                             jnp.full((_NPB - N,), B, jnp.int32)])
    bshift = jnp.concatenate([bflat[1:], jnp.full((1,), B, jnp.int32)])
    batch3 = bflat.reshape(_TG, 1, _TBLK)
    bnext3 = bshift.reshape(_TG, 1, _TBLK)
    last_lt = _last_gather(batch3, bnext3, item_lt)

    wsc = jnp.pad(W_score[0], (0, W - H)).reshape(1, W)
    bsc = b_score.reshape(1, 1)
    att, smax = _att_smax(item_lt, batch3, last_lt, wsc, bsc)
    wsum, sessu = _pool(att, batch3, item_lt, smax)

    b_fc2 = b_fc.reshape(1, -1)
    return _scores(sessu, wsum, W_fc, b_fc2)


# speedup vs baseline: 3.5442x; 1.0394x over previous
import functools

import jax
import jax.numpy as jnp
from jax import lax
from jax.experimental import pallas as pl
from jax.experimental.pallas import tpu as pltpu
from jax.experimental.pallas import tpu_sc as plsc

_N = 50000
_E = 800000
_B = 512
_H = 100
_W = 112
_NP = 50048
_NPB = 50176
_EPAD = 802816
_EROWS = 6272
_RPT = 392
_EB = 28
_ZROW = _NP // 16
_TBLK = 1024
_TG = 49
_NI_BLK = 512
_IDXROWS = 391


_IDXP = 416
_GOUT = _IDXP * 128


def _emb_body(t0, t1, t2, t3, t4, i0, i1, i2, i3, i4,
              o0, o1, o2, o3, o4, idxb, stage, sem):
    c = lax.axis_index("c")
    s = lax.axis_index("s")
    w = c * 16 + s
    r0 = w * 13
    for t in range(5):
        tab = (t0, t1, t2, t3, t4)[t]
        idx2 = (i0, i1, i2, i3, i4)[t]
        out = (o0, o1, o2, o3, o4)[t]
        pltpu.sync_copy(idx2.at[pl.ds(r0, 13)], idxb)
        hs = [pltpu.async_copy(tab.at[idxb.at[j]], stage.at[j], sem)
              for j in range(13)]
        for h in hs:
            h.wait()
        for j in range(13):
            pltpu.sync_copy(stage.at[j], out.at[pl.ds((r0 + j) * 128, 128)])


def _emb_gather(tables, idxs):
    mesh = plsc.VectorSubcoreMesh(core_axis_name="c", subcore_axis_name="s")
    outs = [jax.ShapeDtypeStruct((_GOUT, 32), jnp.float32) for _ in range(5)]
    kern = functools.partial(
        pl.kernel,
        out_type=outs,
        mesh=mesh,
        scratch_types=[
            pltpu.VMEM((13, 128), jnp.int32),
            pltpu.VMEM((13, 128, 32), jnp.float32),
            pltpu.SemaphoreType.DMA,
        ],
        compiler_params=pltpu.CompilerParams(use_tc_tiling_on_sc=False),
    )(_emb_body)
    return kern(*tables, *idxs)


def _proj_body(price, g0, g1, g2, g3, g4, wg, wp, brow, *outs):
    xg = jnp.concatenate([g0[...], g1[...], g2[...], g3[...], g4[...]], axis=1)
    node = (jnp.dot(xg, wg[...], preferred_element_type=jnp.float32)
            + price[...] * wp[...] + brow[...])
    for t in range(7):
        outs[t][...] = node[:, 16 * t:16 * (t + 1)]


def _proj(price_pad, gs, wg, wp, brow):
    outs = [jax.ShapeDtypeStruct((_NP, 16), jnp.float32) for _ in range(7)]
    return pl.pallas_call(
        _proj_body,
        grid=(_TG,),
        in_specs=[
            pl.BlockSpec((_TBLK, 1), lambda i: (i, 0)),
            *[pl.BlockSpec((_TBLK, 32), lambda i: (i, 0)) for _ in range(5)],
            pl.BlockSpec((160, _W), lambda i: (0, 0)),
            pl.BlockSpec((1, _W), lambda i: (0, 0)),
            pl.BlockSpec((1, _W), lambda i: (0, 0)),
        ],
        out_specs=[pl.BlockSpec((_TBLK, 16), lambda i: (i, 0))
                   for _ in range(7)],
        out_shape=outs,
    )(price_pad, *gs, wg, wp, brow)


def _edge_body(n0h, n1h, n2h, n3h, n4h, n5h, n6h, srch, dsth,
               o0, o1, o2, o3, o4, o5, o6,
               idx_s, idx_d, stage, zbuf, acc, gsem, ssem):
    c = lax.axis_index("c")
    s = lax.axis_index("s")

    @pl.loop(0, 136)
    def _z(r):
        zbuf[r, pl.ds(0, 16)] = jnp.zeros((16,), jnp.float32)

    chunks = [(n0h, o0), (n1h, o1), (n2h, o2), (n3h, o3), (n4h, o4),
              (n5h, o5), (n6h, o6)]
    for fc in range(7):
        nh, oh = chunks[fc]

        @pl.when(c == fc // 4)
        def _pass(nh=nh, oh=oh):
            @pl.loop(0, 23)
            def _zero(i):
                pltpu.sync_copy(zbuf, acc.at[pl.ds(s * _ZROW + i * 136, 136)])

            plsc.subcore_barrier()

            @pl.loop(0, _RPT // _EB)
            def _outer(it):
                r0 = s * _RPT + it * _EB
                pltpu.sync_copy(srch.at[pl.ds(r0, _EB)], idx_s)
                pltpu.sync_copy(dsth.at[pl.ds(r0, _EB)], idx_d)
                gh = [pltpu.async_copy(nh.at[idx_s.at[j]], stage.at[j], gsem)
                      for j in range(_EB)]
                for h in gh:
                    h.wait()
                sh = [pltpu.async_copy(stage.at[j], acc.at[idx_d.at[j]], ssem,
                                       add=True)
                      for j in range(_EB)]
                for h in sh:
                    h.wait()

            plsc.subcore_barrier()
            pltpu.sync_copy(acc.at[pl.ds(s * _ZROW, _ZROW)],
                            oh.at[pl.ds(s * _ZROW, _ZROW)])


def _edge_msg(nchunks, src2d, dst2d):
    mesh = plsc.VectorSubcoreMesh(core_axis_name="c", subcore_axis_name="s")
    outs = [jax.ShapeDtypeStruct((_NP, 16), jnp.float32) for _ in range(7)]
    kern = functools.partial(
        pl.kernel,
        out_type=outs,
        mesh=mesh,
        scratch_types=[
            pltpu.VMEM((_EB, 128), jnp.int32),
            pltpu.VMEM((_EB, 128), jnp.int32),
            pltpu.VMEM((_EB, 128, 16), jnp.float32),
            pltpu.VMEM((136, 16), jnp.float32),
            pltpu.VMEM_SHARED((_NP, 16), jnp.float32),
            pltpu.SemaphoreType.DMA,
            pltpu.SemaphoreType.DMA,
        ],
        compiler_params=pltpu.CompilerParams(use_tc_tiling_on_sc=False),
    )(_edge_body)
    return kern(*nchunks, src2d, dst2d)


def _gru_body(m0, m1, m2, m3, m4, m5, m6, n0, n1, n2, n3, n4, n5, n6,
              wih, whh, bi, bh, watt, batt, ilt_out):
    msgf = jnp.concatenate([m[...] for m in (m0, m1, m2, m3, m4, m5, m6)],
                           axis=1)
    h = jnp.concatenate([n[...] for n in (n0, n1, n2, n3, n4, n5, n6)],
                        axis=1)
    cnt = jnp.maximum(msgf[:, 100:101], 1.0)
    x = msgf / cnt
    gi = jnp.dot(x, wih[...], preferred_element_type=jnp.float32) + bi[...]
    gh = jnp.dot(h, whh[...], preferred_element_type=jnp.float32) + bh[...]
    r = jax.nn.sigmoid(gi[:, 0:_W] + gh[:, 0:_W])
    z = jax.nn.sigmoid(gi[:, _W:2 * _W] + gh[:, _W:2 * _W])
    n = jnp.tanh(gi[:, 2 * _W:3 * _W] + r * gh[:, 2 * _W:3 * _W])
    hn = (1.0 - z) * n + z * h
    ilt_out[...] = (jnp.dot(hn, watt[...], preferred_element_type=jnp.float32)
                    + batt[...])


def _gru_att(mchunks, nchunks, wih, whh, bi, bh, watt, batt):
    return pl.pallas_call(
        _gru_body,
        grid=(_TG,),
        in_specs=[
            *[pl.BlockSpec((_TBLK, 16), lambda i: (i, 0)) for _ in range(14)],
            pl.BlockSpec((_W, 3 * _W), lambda i: (0, 0)),
            pl.BlockSpec((_W, 3 * _W), lambda i: (0, 0)),
            pl.BlockSpec((1, 3 * _W), lambda i: (0, 0)),
            pl.BlockSpec((1, 3 * _W), lambda i: (0, 0)),
            pl.BlockSpec((_W, _W), lambda i: (0, 0)),
            pl.BlockSpec((1, _W), lambda i: (0, 0)),
        ],
        out_specs=pl.BlockSpec((_TBLK, _W), lambda i: (i, 0)),
        out_shape=jax.ShapeDtypeStruct((_NP, _W), jnp.float32),
    )(*mchunks, *nchunks, wih, whh, bi, bh, watt, batt)


def _lastlt_body(ilt, b3, bn3, out, acc):
    i = pl.program_id(0)
    bvec = b3[0, 0, :]
    nvec = bn3[0, 0, :]
    oh = bvec[:, None] == lax.broadcasted_iota(jnp.int32, (_TBLK, _B), 1)
    m = bvec[:, None] != nvec[:, None]
    ohm = jnp.logical_and(oh, m).astype(jnp.float32)
    part = lax.dot_general(ohm, ilt[...], (((0,), (0,)), ((), ())),
                           preferred_element_type=jnp.float32)

    @pl.when(i == 0)
    def _():
        acc[...] = jnp.zeros((_B, _W), jnp.float32)

    acc[...] += part

    @pl.when(i == _TG - 1)
    def _():
        out[...] = acc[...]


def _last_gather(batch3, bnext3, item_lt):
    return pl.pallas_call(
        _lastlt_body,
        grid=(_TG,),
        in_specs=[
            pl.BlockSpec((_TBLK, _W), lambda i: (i, 0)),
            pl.BlockSpec((1, 1, _TBLK), lambda i: (i, 0, 0)),
            pl.BlockSpec((1, 1, _TBLK), lambda i: (i, 0, 0)),
        ],
        out_specs=pl.BlockSpec((_B, _W), lambda i: (0, 0)),
        out_shape=jax.ShapeDtypeStruct((_B, _W), jnp.float32),
        scratch_shapes=[pltpu.VMEM((_B, _W), jnp.float32)],
    )(item_lt, batch3, bnext3)


def _att_body(ilt, b3, llt, wsc, bsc, att_out, smax_out, acc):
    i = pl.program_id(0)
    bvec = b3[0, 0, :]
    oh = bvec[:, None] == lax.broadcasted_iota(jnp.int32, (_TBLK, _B), 1)
    ohf = oh.astype(jnp.float32)
    expand = jnp.dot(ohf, llt[...], preferred_element_type=jnp.float32)
    sg = jax.nn.sigmoid(ilt[...] + expand)
    att = jnp.sum(sg * wsc[...], axis=1, keepdims=True) + bsc[0, 0]
    att_out[...] = att
    rows = i * _TBLK + lax.broadcasted_iota(jnp.int32, (_TBLK, 1), 0)
    valid = rows < _N
    attm = jnp.where(jnp.logical_and(oh, valid), att, -1e30)
    part = jnp.max(attm, axis=0, keepdims=True)

    @pl.when(i == 0)
    def _():
        acc[...] = jnp.full((1, _B), -1e30, jnp.float32)

    acc[...] = jnp.maximum(acc[...], part)

    @pl.when(i == _TG - 1)
    def _():
        smax_out[...] = acc[...]


def _att_smax(item_lt, batch3, last_lt, wsc, bsc):
    return pl.pallas_call(
        _att_body,
        grid=(_TG,),
        in_specs=[
            pl.BlockSpec((_TBLK, _W), lambda i: (i, 0)),
            pl.BlockSpec((1, 1, _TBLK), lambda i: (i, 0, 0)),
            pl.BlockSpec((_B, _W), lambda i: (0, 0)),
            pl.BlockSpec((1, _W), lambda i: (0, 0)),
            pl.BlockSpec((1, 1), lambda i: (0, 0)),
        ],
        out_specs=[
            pl.BlockSpec((_TBLK, 1), lambda i: (i, 0)),
            pl.BlockSpec((1, _B), lambda i: (0, 0)),
        ],
        out_shape=[
            jax.ShapeDtypeStruct((_NPB, 1), jnp.float32),
            jax.ShapeDtypeStruct((1, _B), jnp.float32),
        ],
        scratch_shapes=[pltpu.VMEM((1, _B), jnp.float32)],
    )(item_lt, batch3, last_lt, wsc, bsc)


def _pool_body(att, b3, ilt, smax, wsum_out, sessu_out, accw, accs):
    i = pl.program_id(0)
    bvec = b3[0, 0, :]
    oh = bvec[:, None] == lax.broadcasted_iota(jnp.int32, (_TBLK, _B), 1)
    ohf = oh.astype(jnp.float32)
    rows = i * _TBLK + lax.broadcasted_iota(jnp.int32, (_TBLK, 1), 0)
    valid = rows < _N
    sm_exp = jnp.sum(ohf * smax[...], axis=1, keepdims=True)
    e = jnp.where(valid, jnp.exp(att[...] - sm_exp), 0.0)
    pw = lax.dot_general(ohf, e, (((0,), (0,)), ((), ())),
                         preferred_element_type=jnp.float32)
    ps = lax.dot_general(ohf, e * ilt[...], (((0,), (0,)), ((), ())),
                         preferred_element_type=jnp.float32)

    @pl.when(i == 0)
    def _():
        accw[...] = jnp.zeros((_B, 1), jnp.float32)
        accs[...] = jnp.zeros((_B, _W), jnp.float32)

    accw[...] += pw
    accs[...] += ps

    @pl.when(i == _TG - 1)
    def _():
        wsum_out[...] = accw[...]
        sessu_out[...] = accs[...]


def _pool(att, batch3, item_lt, smax):
    return pl.pallas_call(
        _pool_body,
        grid=(_TG,),
        in_specs=[
            pl.BlockSpec((_TBLK, 1), lambda i: (i, 0)),
            pl.BlockSpec((1, 1, _TBLK), lambda i: (i, 0, 0)),
            pl.BlockSpec((_TBLK, _W), lambda i: (i, 0)),
            pl.BlockSpec((1, _B), lambda i: (0, 0)),
        ],
        out_specs=[
            pl.BlockSpec((_B, 1), lambda i: (0, 0)),
            pl.BlockSpec((_B, _W), lambda i: (0, 0)),
        ],
        out_shape=[
            jax.ShapeDtypeStruct((_B, 1), jnp.float32),
            jax.ShapeDtypeStruct((_B, _W), jnp.float32),
        ],
        scratch_shapes=[
            pltpu.VMEM((_B, 1), jnp.float32),
            pltpu.VMEM((_B, _W), jnp.float32),
        ],
    )(att, batch3, item_lt, smax)


def _scores_body(sessu, wsum, wfc, bfc, out):
    sess = sessu[...] / (wsum[...] + 1e-16)
    out[...] = (lax.dot_general(sess[:, :_H], wfc[...],
                                (((1,), (1,)), ((), ())),
                                preferred_element_type=jnp.float32)
                + bfc[...])


def _scores(sessu, wsum, w_fc, b_fc2):
    NI = w_fc.shape[0]
    return pl.pallas_call(
        _scores_body,
        grid=(pl.cdiv(NI, _NI_BLK),),
        in_specs=[
            pl.BlockSpec((_B, _W), lambda i: (0, 0)),
            pl.BlockSpec((_B, 1), lambda i: (0, 0)),
            pl.BlockSpec((_NI_BLK, _H), lambda i: (i, 0)),
            pl.BlockSpec((1, _NI_BLK), lambda i: (0, i)),
        ],
        out_specs=pl.BlockSpec((_B, _NI_BLK), lambda i: (0, i)),
        out_shape=jax.ShapeDtypeStruct((_B, NI), jnp.float32),
    )(sessu, wsum, w_fc, b_fc2)


def kernel(price_tensor, category, sub_category, element, brand, product_id_remapped,
           edge_index, batch, cat_emb, sub_emb, elem_emb, brand_emb, item_emb,
           W_msg, b_msg, W_ih, W_hh, b_ih, b_hh, W_att, b_att, W_score, b_score,
           W_fc, b_fc):
    N, B, H, W = _N, _B, _H, _W
    f32 = jnp.float32

    tables = [jnp.pad(t, ((0, 0), (0, 32 - 25)))
              for t in (cat_emb, sub_emb, elem_emb, brand_emb, item_emb)]
    idxs = [jnp.pad(ix, (0, _GOUT - N)).reshape(_IDXP, 128)
            for ix in (category, sub_category, element, brand,
                       product_id_remapped)]
    gs = _emb_gather(tables, idxs)

    wg = jnp.zeros((160, W), f32)
    for t in range(5):
        wg = wg.at[32 * t:32 * t + 25, :H].set(W_msg[:, 1 + 25 * t:26 + 25 * t].T)
    wp = jnp.pad(W_msg[:, 0], (0, W - H)).reshape(1, W)
    brow = jnp.concatenate([b_msg, jnp.ones((1,), f32),
                            jnp.zeros((W - H - 1,), f32)]).reshape(1, W)
    price_pad = jnp.pad(price_tensor, ((0, _NP - N), (0, 0)))
    nchunks = _proj(price_pad, gs, wg, wp, brow)

    src = edge_index[0]
    dst = edge_index[1]
    src_p = jnp.concatenate(
        [src, jnp.full((_EPAD - _E,), N, jnp.int32)]).reshape(_EROWS, 128)
    dst_p = jnp.concatenate(
        [dst, jnp.full((_EPAD - _E,), N, jnp.int32)]).reshape(_EROWS, 128)
    mchunks = _edge_msg(nchunks, src_p, dst_p)

    wih = jnp.zeros((W, 3 * W), f32)
    whh = jnp.zeros((W, 3 * W), f32)
    bi = jnp.zeros((1, 3 * W), f32)
    bh = jnp.zeros((1, 3 * W), f32)
    for g in range(3):
        wih = wih.at[:H, W * g:W * g + H].set(W_ih[H * g:H * (g + 1), :].T)
        whh = whh.at[:H, W * g:W * g + H].set(W_hh[H * g:H * (g + 1), :].T)
        bi = bi.at[0, W * g:W * g + H].set(b_ih[H * g:H * (g + 1)])
        bh = bh.at[0, W * g:W * g + H].set(b_hh[H * g:H * (g + 1)])
    watt = jnp.pad(W_att.T, ((0, W - H), (0, W - H)))
    batt = jnp.pad(b_att, (0, W - H)).reshape(1, W)
    item_lt = _gru_att(mchunks, nchunks, wih, whh, bi, bh, watt, batt)

    bflat = jnp.concatenate([batch.astype(jnp.int32),
                             jnp.full((_NPB - N,), B, jnp.int32)])
    bshift = jnp.concatenate([bflat[1:], jnp.full((1,), B, jnp.int32)])
    batch3 = bflat.reshape(_TG, 1, _TBLK)
    bnext3 = bshift.reshape(_TG, 1, _TBLK)
    last_lt = _last_gather(batch3, bnext3, item_lt)

    wsc = jnp.pad(W_score[0], (0, W - H)).reshape(1, W)
    bsc = b_score.reshape(1, 1)
    att, smax = _att_smax(item_lt, batch3, last_lt, wsc, bsc)
    wsum, sessu = _pool(att, batch3, item_lt, smax)

    b_fc2 = b_fc.reshape(1, -1)
    return _scores(sessu, wsum, W_fc, b_fc2)
